# Initial kernel scaffold; baseline (speedup 1.0000x reference)
import jax
import jax.numpy as jnp
from jax.experimental import pallas as pl


def kernel(z, in_proj_v, in_proj_g, in_proj_b, out_proj_v, out_proj_g, out_proj_b, codebook):
    raise NotImplementedError("write your pallas kernel here")


# speedup vs baseline: 1.1126x; 1.1126x over previous
import functools

import jax
import jax.numpy as jnp
from jax import lax
from jax.experimental import pallas as pl
from jax.experimental.pallas import tpu as pltpu
from jax.experimental.pallas import tpu_sc as plsc

B, D_IN, T = 8, 768, 2048
K, D_CODE = 8192, 64

TB = 256
NT = T // TB
KC = 2048
NKC = K // KC

NW = 32
BPW = (B * T) // NW
GC = 128
NGC = BPW // GC
DPAD = 128


def _prep_body(in_v_ref, in_g_ref, out_v_ref, out_g_ref, cb_ref,
               w_in_ref, w_out_ref, cn_ref, c2_ref):
    v = in_v_ref[...]
    n = jnp.sqrt(jnp.sum(v * v, axis=1, keepdims=True))
    w_in_ref[...] = in_g_ref[...] * v / n
    u = out_v_ref[...]
    m = jnp.sqrt(jnp.sum(u * u, axis=1, keepdims=True))
    w_out_ref[...] = out_g_ref[...] * u / m
    cb = cb_ref[...]
    nc = jnp.sqrt(jnp.sum(cb * cb, axis=1, keepdims=True))
    cn = cb / jnp.maximum(nc, 1e-12)
    cn_ref[...] = cn
    c2_ref[...] = jnp.sum(cn * cn, axis=1, keepdims=True)


def _prep(in_v, in_g, out_v, out_g, cb):
    return pl.pallas_call(
        _prep_body,
        out_shape=[
            jax.ShapeDtypeStruct((D_CODE, D_IN), jnp.float32),
            jax.ShapeDtypeStruct((D_IN, D_CODE), jnp.float32),
            jax.ShapeDtypeStruct((K, D_CODE), jnp.float32),
            jax.ShapeDtypeStruct((K, 1), jnp.float32),
        ],
    )(in_v, in_g, out_v, out_g, cb)


def _enc_body(z_ref, w_ref, b_ref, cn_ref, c2_ref, ze_ref, idx_ref):
    ze = jnp.dot(w_ref[...], z_ref[0], preferred_element_type=jnp.float32)
    ze = ze + b_ref[...]
    ze_ref[0] = ze
    enc = ze.T
    n = jnp.sqrt(jnp.sum(enc * enc, axis=1, keepdims=True))
    e = enc / jnp.maximum(n, 1e-12)
    e2 = jnp.sum(e * e, axis=1, keepdims=True)
    halves = []
    for h in range(2):
        best_v = jnp.full((TB, 1), -jnp.inf, dtype=jnp.float32)
        best_i = jnp.zeros((TB, 1), dtype=jnp.int32)
        for jj in range(NKC // 2):
            j = h * (NKC // 2) + jj
            s = jnp.dot(e, cn_ref[:, j * KC:(j + 1) * KC],
                        preferred_element_type=jnp.float32)
            d = e2 - 2.0 * s + c2_ref[:, j * KC:(j + 1) * KC]
            negd = -d
            m = jnp.max(negd, axis=1, keepdims=True)
            ii = lax.broadcasted_iota(jnp.int32, (TB, KC), 1)
            a = jnp.min(jnp.where(negd == m, ii, K), axis=1, keepdims=True)
            a = a + j * KC
            upd = m > best_v
            best_i = jnp.where(upd, a, best_i)
            best_v = jnp.where(upd, m, best_v)
        halves.append((best_v, best_i))
    (m0, i0), (m1, i1) = halves
    u = lax.bitcast_convert_type(m0, jnp.int32)
    lsb = lax.shift_right_logical(u, 16) & 1
    uq = (u + 0x7FFF + lsb) & jnp.int32(-65536)
    m0q = lax.bitcast_convert_type(uq, jnp.float32)
    take = m1 > m0q
    idx_ref[0] = jnp.where(take, i1, i0)


def _encode(z, w_in, b_in, cn_t, c2_t):
    return pl.pallas_call(
        _enc_body,
        grid=(B, NT),
        in_specs=[
            pl.BlockSpec((1, D_IN, TB), lambda b, t: (b, 0, t)),
            pl.BlockSpec((D_CODE, D_IN), lambda b, t: (0, 0)),
            pl.BlockSpec((D_CODE, 1), lambda b, t: (0, 0)),
            pl.BlockSpec((D_CODE, K), lambda b, t: (0, 0)),
            pl.BlockSpec((1, K), lambda b, t: (0, 0)),
        ],
        out_specs=[
            pl.BlockSpec((1, D_CODE, TB), lambda b, t: (b, 0, t)),
            pl.BlockSpec((1, TB, 1), lambda b, t: (b * NT + t, 0, 0)),
        ],
        out_shape=[
            jax.ShapeDtypeStruct((B, D_CODE, T), jnp.float32),
            jax.ShapeDtypeStruct((B * NT, TB, 1), jnp.int32),
        ],
    )(z, w_in, b_in, cn_t, c2_t)


def _gather_body(table_hbm, idx_hbm, out_hbm, idx_v, rows_v, sem):
    wid = lax.axis_index("s") * 2 + lax.axis_index("c")
    base = wid * BPW
    pltpu.sync_copy(idx_hbm.at[pl.ds(base, BPW)], idx_v)
    copies = []
    for j in range(NGC):
        copies.append(pltpu.async_copy(
            table_hbm.at[idx_v.at[pl.ds(j * GC, GC)]],
            rows_v.at[pl.ds(j * GC, GC)], sem))
    for c in copies:
        c.wait()
    pltpu.sync_copy(rows_v, out_hbm.at[pl.ds(base, BPW)])


def _gather_rows(table_pad, idx_flat):
    mesh = plsc.VectorSubcoreMesh(core_axis_name="c", subcore_axis_name="s")
    fn = pl.kernel(
        _gather_body,
        mesh=mesh,
        out_type=jax.ShapeDtypeStruct((B * T, DPAD), jnp.float32),
        scratch_types=[
            pltpu.VMEM((BPW,), jnp.int32),
            pltpu.VMEM((BPW, DPAD), jnp.float32),
            pltpu.SemaphoreType.DMA,
        ],
    )
    return fn(table_pad, idx_flat)


def _dec_body(ze_ref, zq_ref, w_ref, b_ref, out_ref, loss_ref):
    t = pl.program_id(1)
    ze = ze_ref[0]
    zq = zq_ref[:, :D_CODE].T
    st = ze + (zq - ze)
    out = jnp.dot(w_ref[...], st, preferred_element_type=jnp.float32)
    out_ref[0] = out + b_ref[...]
    diff = ze - zq
    psum = jnp.sum(diff * diff)

    @pl.when(t == 0)
    def _():
        loss_ref[...] = jnp.zeros_like(loss_ref)

    loss_ref[...] += psum


def _decode(z_e, z_q_pad, w_out, b_out):
    return pl.pallas_call(
        _dec_body,
        grid=(B, NT),
        in_specs=[
            pl.BlockSpec((1, D_CODE, TB), lambda b, t: (b, 0, t)),
            pl.BlockSpec((TB, DPAD), lambda b, t: (b * NT + t, 0)),
            pl.BlockSpec((D_IN, D_CODE), lambda b, t: (0, 0)),
            pl.BlockSpec((D_IN, 1), lambda b, t: (0, 0)),
        ],
        out_specs=[
            pl.BlockSpec((1, D_IN, TB), lambda b, t: (b, 0, t)),
            pl.BlockSpec((1, 1, 128), lambda b, t: (b, 0, 0)),
        ],
        out_shape=[
            jax.ShapeDtypeStruct((B, D_IN, T), jnp.float32),
            jax.ShapeDtypeStruct((B, 1, 128), jnp.float32),
        ],
    )(z_e, z_q_pad, w_out, b_out)


def kernel(z, in_proj_v, in_proj_g, in_proj_b,
           out_proj_v, out_proj_g, out_proj_b, codebook):
    w_in, w_out, cn, c2 = _prep(in_proj_v, in_proj_g,
                                out_proj_v, out_proj_g, codebook)
    cn_t = cn.T
    c2_t = c2.reshape(1, K)
    b_in = in_proj_b.reshape(D_CODE, 1)
    b_out = out_proj_b.reshape(D_IN, 1)
    z_e, idx3 = _encode(z, w_in, b_in, cn_t, c2_t)
    indices = idx3.reshape(B, T)
    table_pad = jnp.pad(codebook, ((0, 0), (0, DPAD - D_CODE)))
    z_q_pad = _gather_rows(table_pad, indices.reshape(B * T))
    out, loss3 = _decode(z_e, z_q_pad, w_out, b_out)
    loss = loss3[:, 0, 0] / (D_CODE * T)
    return (out, loss, loss, indices, z_e)


# speedup vs baseline: 1.2012x; 1.0796x over previous
import functools

import jax
import jax.numpy as jnp
from jax import lax
from jax.experimental import pallas as pl
from jax.experimental.pallas import tpu as pltpu
from jax.experimental.pallas import tpu_sc as plsc

B, D_IN, T = 8, 768, 2048
K, D_CODE = 8192, 64

TB = 512
NT = T // TB
KC = 2048
NKC = K // KC

NW = 32
BPW = (B * T) // NW
GC = 128
NGC = BPW // GC
DPAD = 128


def _prep_body(in_v_ref, in_g_ref, out_v_ref, out_g_ref, cb_ref,
               w_in_ref, w_out_ref, cn_ref, c2_ref):
    v = in_v_ref[...]
    n = jnp.sqrt(jnp.sum(v * v, axis=1, keepdims=True))
    w_in_ref[...] = in_g_ref[...] * v / n
    u = out_v_ref[...]
    m = jnp.sqrt(jnp.sum(u * u, axis=1, keepdims=True))
    w_out_ref[...] = out_g_ref[...] * u / m
    cb = cb_ref[...]
    nc = jnp.sqrt(jnp.sum(cb * cb, axis=1, keepdims=True))
    cn = cb / jnp.maximum(nc, 1e-12)
    cn_ref[...] = cn
    c2_ref[...] = jnp.sum(cn * cn, axis=1, keepdims=True)


def _prep(in_v, in_g, out_v, out_g, cb):
    return pl.pallas_call(
        _prep_body,
        out_shape=[
            jax.ShapeDtypeStruct((D_CODE, D_IN), jnp.float32),
            jax.ShapeDtypeStruct((D_IN, D_CODE), jnp.float32),
            jax.ShapeDtypeStruct((K, D_CODE), jnp.float32),
            jax.ShapeDtypeStruct((K, 1), jnp.float32),
        ],
    )(in_v, in_g, out_v, out_g, cb)


def _enc_body(z_ref, w_ref, b_ref, cn_ref, c2_ref, ze_ref, idx_ref):
    ze = jnp.dot(w_ref[...], z_ref[0], preferred_element_type=jnp.float32)
    ze = ze + b_ref[...]
    ze_ref[0] = ze
    enc = ze.T
    n = jnp.sqrt(jnp.sum(enc * enc, axis=1, keepdims=True))
    e = enc / jnp.maximum(n, 1e-12)
    e2 = jnp.sum(e * e, axis=1, keepdims=True)
    halves = []
    for h in range(2):
        best_v = jnp.full((TB, 1), -jnp.inf, dtype=jnp.float32)
        best_i = jnp.zeros((TB, 1), dtype=jnp.int32)
        for jj in range(NKC // 2):
            j = h * (NKC // 2) + jj
            s = jnp.dot(e, cn_ref[:, j * KC:(j + 1) * KC],
                        preferred_element_type=jnp.float32)
            d = e2 - 2.0 * s + c2_ref[:, j * KC:(j + 1) * KC]
            negd = -d
            m = jnp.max(negd, axis=1, keepdims=True)
            ii = lax.broadcasted_iota(jnp.int32, (TB, KC), 1)
            a = jnp.min(jnp.where(negd == m, ii, K), axis=1, keepdims=True)
            a = a + j * KC
            upd = m > best_v
            best_i = jnp.where(upd, a, best_i)
            best_v = jnp.where(upd, m, best_v)
        halves.append((best_v, best_i))
    (m0, i0), (m1, i1) = halves
    u = lax.bitcast_convert_type(m0, jnp.int32)
    lsb = lax.shift_right_logical(u, 16) & 1
    uq = (u + 0x7FFF + lsb) & jnp.int32(-65536)
    m0q = lax.bitcast_convert_type(uq, jnp.float32)
    take = m1 > m0q
    idx_ref[0] = jnp.where(take, i1, i0)


def _encode(z, w_in, b_in, cn_t, c2_t):
    return pl.pallas_call(
        _enc_body,
        grid=(B, NT),
        in_specs=[
            pl.BlockSpec((1, D_IN, TB), lambda b, t: (b, 0, t)),
            pl.BlockSpec((D_CODE, D_IN), lambda b, t: (0, 0)),
            pl.BlockSpec((D_CODE, 1), lambda b, t: (0, 0)),
            pl.BlockSpec((D_CODE, K), lambda b, t: (0, 0)),
            pl.BlockSpec((1, K), lambda b, t: (0, 0)),
        ],
        out_specs=[
            pl.BlockSpec((1, D_CODE, TB), lambda b, t: (b, 0, t)),
            pl.BlockSpec((1, TB, 1), lambda b, t: (b * NT + t, 0, 0)),
        ],
        out_shape=[
            jax.ShapeDtypeStruct((B, D_CODE, T), jnp.float32),
            jax.ShapeDtypeStruct((B * NT, TB, 1), jnp.int32),
        ],
    )(z, w_in, b_in, cn_t, c2_t)


def _gather_body(table_hbm, idx_hbm, out_hbm, idx_v, rows_v, sem):
    wid = lax.axis_index("s") * 2 + lax.axis_index("c")
    base = wid * BPW
    pltpu.sync_copy(idx_hbm.at[pl.ds(base, BPW)], idx_v)
    copies = []
    for j in range(NGC):
        copies.append(pltpu.async_copy(
            table_hbm.at[idx_v.at[pl.ds(j * GC, GC)]],
            rows_v.at[pl.ds(j * GC, GC)], sem))
    for c in copies:
        c.wait()
    pltpu.sync_copy(rows_v, out_hbm.at[pl.ds(base, BPW)])


def _gather_rows(table_pad, idx_flat):
    mesh = plsc.VectorSubcoreMesh(core_axis_name="c", subcore_axis_name="s")
    fn = pl.kernel(
        _gather_body,
        mesh=mesh,
        out_type=jax.ShapeDtypeStruct((B * T, DPAD), jnp.float32),
        scratch_types=[
            pltpu.VMEM((BPW,), jnp.int32),
            pltpu.VMEM((BPW, DPAD), jnp.float32),
            pltpu.SemaphoreType.DMA,
        ],
    )
    return fn(table_pad, idx_flat)


def _dec_body(ze_ref, zq_ref, w_ref, b_ref, out_ref, loss_ref):
    t = pl.program_id(1)
    ze = ze_ref[0]
    zq = zq_ref[:, :D_CODE].T
    st = ze + (zq - ze)
    out = jnp.dot(w_ref[...], st, preferred_element_type=jnp.float32)
    out_ref[0] = out + b_ref[...]
    diff = ze - zq
    psum = jnp.sum(diff * diff)

    @pl.when(t == 0)
    def _():
        loss_ref[...] = jnp.zeros_like(loss_ref)

    loss_ref[...] += psum


def _decode(z_e, z_q_pad, w_out, b_out):
    return pl.pallas_call(
        _dec_body,
        grid=(B, NT),
        in_specs=[
            pl.BlockSpec((1, D_CODE, TB), lambda b, t: (b, 0, t)),
            pl.BlockSpec((TB, DPAD), lambda b, t: (b * NT + t, 0)),
            pl.BlockSpec((D_IN, D_CODE), lambda b, t: (0, 0)),
            pl.BlockSpec((D_IN, 1), lambda b, t: (0, 0)),
        ],
        out_specs=[
            pl.BlockSpec((1, D_IN, TB), lambda b, t: (b, 0, t)),
            pl.BlockSpec((1, 1, 128), lambda b, t: (b, 0, 0)),
        ],
        out_shape=[
            jax.ShapeDtypeStruct((B, D_IN, T), jnp.float32),
            jax.ShapeDtypeStruct((B, 1, 128), jnp.float32),
        ],
    )(z_e, z_q_pad, w_out, b_out)


def kernel(z, in_proj_v, in_proj_g, in_proj_b,
           out_proj_v, out_proj_g, out_proj_b, codebook):
    w_in, w_out, cn, c2 = _prep(in_proj_v, in_proj_g,
                                out_proj_v, out_proj_g, codebook)
    cn_t = cn.T
    c2_t = c2.reshape(1, K)
    b_in = in_proj_b.reshape(D_CODE, 1)
    b_out = out_proj_b.reshape(D_IN, 1)
    z_e, idx3 = _encode(z, w_in, b_in, cn_t, c2_t)
    indices = idx3.reshape(B, T)
    table_pad = jnp.pad(codebook, ((0, 0), (0, DPAD - D_CODE)))
    z_q_pad = _gather_rows(table_pad, indices.reshape(B * T))
    out, loss3 = _decode(z_e, z_q_pad, w_out, b_out)
    loss = loss3[:, 0, 0] / (D_CODE * T)
    return (out, loss, loss, indices, z_e)


# speedup vs baseline: 1.3133x; 1.0933x over previous
import functools

import jax
import jax.numpy as jnp
from jax import lax
from jax.experimental import pallas as pl
from jax.experimental.pallas import tpu as pltpu
from jax.experimental.pallas import tpu_sc as plsc

B, D_IN, T = 8, 768, 2048
K, D_CODE = 8192, 64

TB = 1024
NT = T // TB
KC = 2048
NKC = K // KC

NW = 32
BPW = (B * T) // NW
GC = 128
NGC = BPW // GC
DPAD = 128


def _prep_body(in_v_ref, in_g_ref, out_v_ref, out_g_ref, cb_ref,
               w_in_ref, w_out_ref, cn_ref, c2_ref):
    v = in_v_ref[...]
    n = jnp.sqrt(jnp.sum(v * v, axis=1, keepdims=True))
    w_in_ref[...] = in_g_ref[...] * v / n
    u = out_v_ref[...]
    m = jnp.sqrt(jnp.sum(u * u, axis=1, keepdims=True))
    w_out_ref[...] = out_g_ref[...] * u / m
    cb = cb_ref[...]
    nc = jnp.sqrt(jnp.sum(cb * cb, axis=1, keepdims=True))
    cn = cb / jnp.maximum(nc, 1e-12)
    cn_ref[...] = cn
    c2_ref[...] = jnp.sum(cn * cn, axis=1, keepdims=True)


def _prep(in_v, in_g, out_v, out_g, cb):
    return pl.pallas_call(
        _prep_body,
        out_shape=[
            jax.ShapeDtypeStruct((D_CODE, D_IN), jnp.float32),
            jax.ShapeDtypeStruct((D_IN, D_CODE), jnp.float32),
            jax.ShapeDtypeStruct((K, D_CODE), jnp.float32),
            jax.ShapeDtypeStruct((K, 1), jnp.float32),
        ],
    )(in_v, in_g, out_v, out_g, cb)


def _enc_body(z_ref, w_ref, b_ref, cn_ref, c2_ref, ze_ref, idx_ref):
    ze = jnp.dot(w_ref[...], z_ref[0], preferred_element_type=jnp.float32)
    ze = ze + b_ref[...]
    ze_ref[0] = ze
    enc = ze.T
    n = jnp.sqrt(jnp.sum(enc * enc, axis=1, keepdims=True))
    e = enc / jnp.maximum(n, 1e-12)
    e2 = jnp.sum(e * e, axis=1, keepdims=True)
    halves = []
    for h in range(2):
        best_v = jnp.full((TB, 1), -jnp.inf, dtype=jnp.float32)
        best_i = jnp.zeros((TB, 1), dtype=jnp.int32)
        for jj in range(NKC // 2):
            j = h * (NKC // 2) + jj
            s = jnp.dot(e, cn_ref[:, j * KC:(j + 1) * KC],
                        preferred_element_type=jnp.float32)
            d = e2 - 2.0 * s + c2_ref[:, j * KC:(j + 1) * KC]
            negd = -d
            m = jnp.max(negd, axis=1, keepdims=True)
            ii = lax.broadcasted_iota(jnp.int32, (TB, KC), 1)
            a = jnp.min(jnp.where(negd == m, ii, K), axis=1, keepdims=True)
            a = a + j * KC
            upd = m > best_v
            best_i = jnp.where(upd, a, best_i)
            best_v = jnp.where(upd, m, best_v)
        halves.append((best_v, best_i))
    (m0, i0), (m1, i1) = halves
    u = lax.bitcast_convert_type(m0, jnp.int32)
    lsb = lax.shift_right_logical(u, 16) & 1
    uq = (u + 0x7FFF + lsb) & jnp.int32(-65536)
    m0q = lax.bitcast_convert_type(uq, jnp.float32)
    take = m1 > m0q
    idx_ref[0] = jnp.where(take, i1, i0)


def _encode(z, w_in, b_in, cn_t, c2_t):
    return pl.pallas_call(
        _enc_body,
        grid=(B, NT),
        in_specs=[
            pl.BlockSpec((1, D_IN, TB), lambda b, t: (b, 0, t)),
            pl.BlockSpec((D_CODE, D_IN), lambda b, t: (0, 0)),
            pl.BlockSpec((D_CODE, 1), lambda b, t: (0, 0)),
            pl.BlockSpec((D_CODE, K), lambda b, t: (0, 0)),
            pl.BlockSpec((1, K), lambda b, t: (0, 0)),
        ],
        out_specs=[
            pl.BlockSpec((1, D_CODE, TB), lambda b, t: (b, 0, t)),
            pl.BlockSpec((1, TB, 1), lambda b, t: (b * NT + t, 0, 0)),
        ],
        out_shape=[
            jax.ShapeDtypeStruct((B, D_CODE, T), jnp.float32),
            jax.ShapeDtypeStruct((B * NT, TB, 1), jnp.int32),
        ],
    )(z, w_in, b_in, cn_t, c2_t)


def _gather_body(table_hbm, idx_hbm, out_hbm, idx_v, rows_v, sem):
    wid = lax.axis_index("s") * 2 + lax.axis_index("c")
    base = wid * BPW
    pltpu.sync_copy(idx_hbm.at[pl.ds(base, BPW)], idx_v)
    copies = []
    for j in range(NGC):
        copies.append(pltpu.async_copy(
            table_hbm.at[idx_v.at[pl.ds(j * GC, GC)]],
            rows_v.at[pl.ds(j * GC, GC)], sem))
    for c in copies:
        c.wait()
    pltpu.sync_copy(rows_v, out_hbm.at[pl.ds(base, BPW)])


def _gather_rows(table_pad, idx_flat):
    mesh = plsc.VectorSubcoreMesh(core_axis_name="c", subcore_axis_name="s")
    fn = pl.kernel(
        _gather_body,
        mesh=mesh,
        out_type=jax.ShapeDtypeStruct((B * T, DPAD), jnp.float32),
        scratch_types=[
            pltpu.VMEM((BPW,), jnp.int32),
            pltpu.VMEM((BPW, DPAD), jnp.float32),
            pltpu.SemaphoreType.DMA,
        ],
    )
    return fn(table_pad, idx_flat)


def _dec_body(ze_ref, zq_ref, w_ref, b_ref, out_ref, loss_ref):
    t = pl.program_id(1)
    ze = ze_ref[0]
    zq = zq_ref[:, :D_CODE].T
    st = ze + (zq - ze)
    out = jnp.dot(w_ref[...], st, preferred_element_type=jnp.float32)
    out_ref[0] = out + b_ref[...]
    diff = ze - zq
    psum = jnp.sum(diff * diff)

    @pl.when(t == 0)
    def _():
        loss_ref[...] = jnp.zeros_like(loss_ref)

    loss_ref[...] += psum


def _decode(z_e, z_q_pad, w_out, b_out):
    return pl.pallas_call(
        _dec_body,
        grid=(B, NT),
        in_specs=[
            pl.BlockSpec((1, D_CODE, TB), lambda b, t: (b, 0, t)),
            pl.BlockSpec((TB, DPAD), lambda b, t: (b * NT + t, 0)),
            pl.BlockSpec((D_IN, D_CODE), lambda b, t: (0, 0)),
            pl.BlockSpec((D_IN, 1), lambda b, t: (0, 0)),
        ],
        out_specs=[
            pl.BlockSpec((1, D_IN, TB), lambda b, t: (b, 0, t)),
            pl.BlockSpec((1, 1, 128), lambda b, t: (b, 0, 0)),
        ],
        out_shape=[
            jax.ShapeDtypeStruct((B, D_IN, T), jnp.float32),
            jax.ShapeDtypeStruct((B, 1, 128), jnp.float32),
        ],
    )(z_e, z_q_pad, w_out, b_out)


def kernel(z, in_proj_v, in_proj_g, in_proj_b,
           out_proj_v, out_proj_g, out_proj_b, codebook):
    w_in, w_out, cn, c2 = _prep(in_proj_v, in_proj_g,
                                out_proj_v, out_proj_g, codebook)
    cn_t = cn.T
    c2_t = c2.reshape(1, K)
    b_in = in_proj_b.reshape(D_CODE, 1)
    b_out = out_proj_b.reshape(D_IN, 1)
    z_e, idx3 = _encode(z, w_in, b_in, cn_t, c2_t)
    indices = idx3.reshape(B, T)
    table_pad = jnp.pad(codebook, ((0, 0), (0, DPAD - D_CODE)))
    z_q_pad = _gather_rows(table_pad, indices.reshape(B * T))
    out, loss3 = _decode(z_e, z_q_pad, w_out, b_out)
    loss = loss3[:, 0, 0] / (D_CODE * T)
    return (out, loss, loss, indices, z_e)
